# Initial kernel scaffold; baseline (speedup 1.0000x reference)
import jax
import jax.numpy as jnp
from jax.experimental import pallas as pl


def kernel(queries, keys):
    raise NotImplementedError("write your pallas kernel here")


# speedup vs baseline: 1.4801x; 1.4801x over previous
import functools

import jax
import jax.numpy as jnp
from jax.experimental import pallas as pl
from jax.experimental.pallas import tpu as pltpu

K_TOP = 10
_NEG_INF = float("-inf")
_BIG_I32 = 2**31 - 1


def _topk_of_block(s, col, k):
    vals = []
    idxs = []
    for _ in range(k):
        m = jnp.max(s, axis=1, keepdims=True)
        eq = s == m
        idx = jnp.min(jnp.where(eq, col, _BIG_I32), axis=1, keepdims=True)
        s = jnp.where(col == idx, _NEG_INF, s)
        vals.append(m)
        idxs.append(idx)
    return jnp.concatenate(vals, axis=1), jnp.concatenate(idxs, axis=1)


def _retriever_kernel(n_keys, bk, q_ref, k_ref, sv_ref, si_ref):
    j = pl.program_id(1)

    s = jnp.dot(q_ref[...], k_ref[...].T, preferred_element_type=jnp.float32)
    col = jax.lax.broadcasted_iota(jnp.int32, s.shape, 1) + j * bk
    s = jnp.where(col < n_keys, s, _NEG_INF)

    bv, bi = _topk_of_block(s, col, K_TOP)

    @pl.when(j == 0)
    def _init():
        sv_ref[...] = bv
        si_ref[...] = bi

    @pl.when(j != 0)
    def _merge():
        cv = jnp.concatenate([sv_ref[...], bv], axis=1)
        ci = jnp.concatenate([si_ref[...], bi], axis=1)
        nv = []
        ni = []
        for _ in range(K_TOP):
            m = jnp.max(cv, axis=1, keepdims=True)
            eq = cv == m
            idx = jnp.min(jnp.where(eq, ci, _BIG_I32), axis=1, keepdims=True)
            cv = jnp.where(jnp.logical_and(eq, ci == idx), _NEG_INF, cv)
            nv.append(m)
            ni.append(idx)
        sv_ref[...] = jnp.concatenate(nv, axis=1)
        si_ref[...] = jnp.concatenate(ni, axis=1)


@jax.jit
def kernel(queries, keys):
    n_q, d = queries.shape
    n_keys = keys.shape[0]

    bq = min(n_q, 256)
    bk = 2048
    n_kb = -(-n_keys // bk)
    k_pad = n_kb * bk
    if k_pad != n_keys:
        keys = jnp.pad(keys, ((0, k_pad - n_keys), (0, 0)))

    grid = (n_q // bq, n_kb)
    out_shapes = (
        jax.ShapeDtypeStruct((n_q, K_TOP), jnp.float32),
        jax.ShapeDtypeStruct((n_q, K_TOP), jnp.int32),
    )
    scores, indices = pl.pallas_call(
        functools.partial(_retriever_kernel, n_keys, bk),
        grid=grid,
        in_specs=[
            pl.BlockSpec((bq, d), lambda i, j: (i, 0)),
            pl.BlockSpec((bk, d), lambda i, j: (j, 0)),
        ],
        out_specs=(
            pl.BlockSpec((bq, K_TOP), lambda i, j: (i, 0)),
            pl.BlockSpec((bq, K_TOP), lambda i, j: (i, 0)),
        ),
        out_shape=out_shapes,
        compiler_params=pltpu.CompilerParams(
            dimension_semantics=("parallel", "arbitrary"),
        ),
    )(queries, keys)
    return scores, indices
